# baseline (speedup 1.0000x reference)
import functools

import jax
import jax.numpy as jnp
from jax import lax
from jax.experimental import pallas as pl
from jax.experimental.pallas import tpu as pltpu
from jax.experimental.pallas import tpu_sc as plsc

D = 100
DP = 112
NC = 2
NS = 16
NW = NC * NS
RPS = 128
SPG = 8
GROUP = RPS * SPG
QUART = GROUP // 4


@functools.lru_cache(maxsize=None)
def _build(n_idx, vocab):
    assert n_idx % (NW * GROUP) == 0
    b_per_w = n_idx // NW
    n_groups = b_per_w // GROUP
    mesh = plsc.VectorSubcoreMesh(core_axis_name="c", subcore_axis_name="s")

    @functools.partial(
        pl.kernel,
        out_type=jax.ShapeDtypeStruct((n_idx, DP), jnp.float32),
        mesh=mesh,
        compiler_params=pltpu.CompilerParams(use_tc_tiling_on_sc=False),
        scratch_types=[
            pltpu.VMEM((SPG, RPS), jnp.int32),
            pltpu.VMEM((QUART, DP), jnp.float32),
            pltpu.VMEM((QUART, DP), jnp.float32),
            pltpu.SemaphoreType.DMA,
            pltpu.SemaphoreType.DMA,
            pltpu.SemaphoreType.DMA,
        ],
    )
    def emb(x_hbm, table_hbm, out_hbm, idx_v, rows0, rows1, gsem, wsem0, wsem1):
        wid = lax.axis_index("s") * NC + lax.axis_index("c")
        base = wid * b_per_w
        base_row = base // RPS

        def drain_write(wsem):
            pltpu.make_async_copy(
                rows0, out_hbm.at[pl.ds(0, QUART)], wsem
            ).wait()

        def do_quarter(g, q, rbuf, wsem):
            copies = []
            for j in range(2):
                copies.append(
                    pltpu.async_copy(
                        table_hbm.at[idx_v.at[2 * q + j]],
                        rbuf.at[pl.ds(j * RPS, RPS)],
                        gsem,
                    )
                )
            for c in copies:
                c.wait()
            obase = pl.multiple_of(base + g * GROUP + q * QUART, QUART)
            pltpu.async_copy(rbuf, out_hbm.at[pl.ds(obase, QUART)], wsem)

        def body(g, carry):
            row0 = pl.multiple_of(base_row + g * SPG, SPG)
            pltpu.sync_copy(x_hbm.at[pl.ds(row0, SPG)], idx_v)
            for q, rbuf, wsem in ((0, rows0, wsem0), (1, rows1, wsem1)):
                @pl.when(g >= 1)
                def _():
                    drain_write(wsem)

                do_quarter(g, q, rbuf, wsem)
            for q, rbuf, wsem in ((2, rows0, wsem0), (3, rows1, wsem1)):
                drain_write(wsem)
                do_quarter(g, q, rbuf, wsem)
            return carry

        lax.fori_loop(0, n_groups, body, 0)
        drain_write(wsem0)
        drain_write(wsem1)

    return emb


def kernel(x, table):
    b, h = x.shape
    n_idx = b * h
    xf = x.astype(jnp.int32).reshape(n_idx // RPS, RPS)
    tpad = jnp.pad(table, ((0, 0), (0, DP - D)))
    out = _build(n_idx, table.shape[0])(xf, tpad)
    return out[:, :D].reshape(b, h, D)


# speedup vs baseline: 1.5958x; 1.5958x over previous
import functools

import jax
import jax.numpy as jnp
from jax import lax
from jax.experimental import pallas as pl
from jax.experimental.pallas import tpu as pltpu
from jax.experimental.pallas import tpu_sc as plsc

D = 100
DP = 128
NC = 2
NS = 16
NW = NC * NS
RPS = 128
SPG = 8
GROUP = RPS * SPG
QUART = GROUP // 4


@functools.lru_cache(maxsize=None)
def _build(n_idx, vocab):
    assert n_idx % (NW * GROUP) == 0
    b_per_w = n_idx // NW
    n_groups = b_per_w // GROUP
    mesh = plsc.VectorSubcoreMesh(core_axis_name="c", subcore_axis_name="s")

    @functools.partial(
        pl.kernel,
        out_type=jax.ShapeDtypeStruct((n_idx, DP), jnp.float32),
        mesh=mesh,
        compiler_params=pltpu.CompilerParams(use_tc_tiling_on_sc=False),
        scratch_types=[
            pltpu.VMEM((SPG, RPS), jnp.int32),
            pltpu.VMEM((QUART, DP), jnp.float32),
            pltpu.VMEM((QUART, DP), jnp.float32),
            pltpu.SemaphoreType.DMA,
            pltpu.SemaphoreType.DMA,
            pltpu.SemaphoreType.DMA,
        ],
    )
    def emb(x_hbm, table_hbm, out_hbm, idx_v, rows0, rows1, gsem, wsem0, wsem1):
        wid = lax.axis_index("s") * NC + lax.axis_index("c")
        base = wid * b_per_w
        base_row = base // RPS

        def drain_write(wsem):
            pltpu.make_async_copy(
                rows0, out_hbm.at[pl.ds(0, QUART)], wsem
            ).wait()

        def do_quarter(g, q, rbuf, wsem):
            copies = []
            for j in range(2):
                copies.append(
                    pltpu.async_copy(
                        table_hbm.at[idx_v.at[2 * q + j]],
                        rbuf.at[pl.ds(j * RPS, RPS)],
                        gsem,
                    )
                )
            for c in copies:
                c.wait()
            obase = pl.multiple_of(base + g * GROUP + q * QUART, QUART)
            pltpu.async_copy(rbuf, out_hbm.at[pl.ds(obase, QUART)], wsem)

        def body(g, carry):
            row0 = pl.multiple_of(base_row + g * SPG, SPG)
            pltpu.sync_copy(x_hbm.at[pl.ds(row0, SPG)], idx_v)
            for q, rbuf, wsem in ((0, rows0, wsem0), (1, rows1, wsem1)):
                @pl.when(g >= 1)
                def _():
                    drain_write(wsem)

                do_quarter(g, q, rbuf, wsem)
            for q, rbuf, wsem in ((2, rows0, wsem0), (3, rows1, wsem1)):
                drain_write(wsem)
                do_quarter(g, q, rbuf, wsem)
            return carry

        lax.fori_loop(0, n_groups, body, 0)
        drain_write(wsem0)
        drain_write(wsem1)

    return emb


def kernel(x, table):
    b, h = x.shape
    n_idx = b * h
    xf = x.astype(jnp.int32).reshape(n_idx // RPS, RPS)
    tpad = jnp.pad(table, ((0, 0), (0, DP - D)))
    out = _build(n_idx, table.shape[0])(xf, tpad)
    return out[:, :D].reshape(b, h, D)


# speedup vs baseline: 1.6153x; 1.0122x over previous
import functools

import jax
import jax.numpy as jnp
from jax import lax
from jax.experimental import pallas as pl
from jax.experimental.pallas import tpu as pltpu
from jax.experimental.pallas import tpu_sc as plsc

D = 100
DP = 128
NC = 2
NS = 16
NW = NC * NS
RPS = 128
SPG = 8
GROUP = RPS * SPG
QUART = GROUP // 4


@functools.lru_cache(maxsize=None)
def _build(n_idx, vocab):
    assert n_idx % (NW * GROUP) == 0
    b_per_w = n_idx // NW
    n_groups = b_per_w // GROUP
    mesh = plsc.VectorSubcoreMesh(core_axis_name="c", subcore_axis_name="s")

    assert n_groups % 2 == 1
    n_pairs = n_groups // 2

    @functools.partial(
        pl.kernel,
        out_type=jax.ShapeDtypeStruct((n_idx, DP), jnp.float32),
        mesh=mesh,
        compiler_params=pltpu.CompilerParams(use_tc_tiling_on_sc=False),
        scratch_types=[
            pltpu.VMEM((SPG, RPS), jnp.int32),
            pltpu.VMEM((SPG, RPS), jnp.int32),
            pltpu.VMEM((QUART, DP), jnp.float32),
            pltpu.VMEM((QUART, DP), jnp.float32),
            pltpu.SemaphoreType.DMA,
            pltpu.SemaphoreType.DMA,
            pltpu.SemaphoreType.DMA,
            pltpu.SemaphoreType.DMA,
            pltpu.SemaphoreType.DMA,
        ],
    )
    def emb(x_hbm, table_hbm, out_hbm, idxA, idxB, rows0, rows1,
            gsem, wsem0, wsem1, isemA, isemB):
        wid = lax.axis_index("s") * NC + lax.axis_index("c")
        base = wid * b_per_w
        base_row = base // RPS

        def fire_idx(g, idx_ref, isem):
            row0 = pl.multiple_of(base_row + g * SPG, SPG)
            pltpu.async_copy(x_hbm.at[pl.ds(row0, SPG)], idx_ref, isem)

        def wait_idx(idx_ref, isem):
            pltpu.make_async_copy(
                x_hbm.at[pl.ds(0, SPG)], idx_ref, isem
            ).wait()

        def drain_write(wsem):
            pltpu.make_async_copy(
                rows0, out_hbm.at[pl.ds(0, QUART)], wsem
            ).wait()

        def do_quarter(g, q, idx_ref, rbuf, wsem):
            copies = []
            for j in range(2):
                copies.append(
                    pltpu.async_copy(
                        table_hbm.at[idx_ref.at[2 * q + j]],
                        rbuf.at[pl.ds(j * RPS, RPS)],
                        gsem,
                    )
                )
            for c in copies:
                c.wait()
            obase = pl.multiple_of(base + g * GROUP + q * QUART, QUART)
            pltpu.async_copy(rbuf, out_hbm.at[pl.ds(obase, QUART)], wsem)

        def do_group(g, idx_ref, skip_first_drains):
            for q, rbuf, wsem in ((0, rows0, wsem0), (1, rows1, wsem1)):
                if skip_first_drains is None:
                    drain_write(wsem)
                else:
                    @pl.when(skip_first_drains)
                    def _():
                        drain_write(wsem)

                do_quarter(g, q, idx_ref, rbuf, wsem)
            for q, rbuf, wsem in ((2, rows0, wsem0), (3, rows1, wsem1)):
                drain_write(wsem)
                do_quarter(g, q, idx_ref, rbuf, wsem)

        fire_idx(0, idxA, isemA)

        def body(p, carry):
            wait_idx(idxA, isemA)
            fire_idx(2 * p + 1, idxB, isemB)
            do_group(2 * p, idxA, p >= 1)
            wait_idx(idxB, isemB)
            fire_idx(2 * p + 2, idxA, isemA)
            do_group(2 * p + 1, idxB, None)
            return carry

        lax.fori_loop(0, n_pairs, body, 0)
        wait_idx(idxA, isemA)
        do_group(n_groups - 1, idxA, None)
        drain_write(wsem0)
        drain_write(wsem1)

    return emb


def kernel(x, table):
    b, h = x.shape
    n_idx = b * h
    xf = x.astype(jnp.int32).reshape(n_idx // RPS, RPS)
    tpad = jnp.pad(table, ((0, 0), (0, DP - D)))
    out = _build(n_idx, table.shape[0])(xf, tpad)
    return out[:, :D].reshape(b, h, D)
